# Initial kernel scaffold; baseline (speedup 1.0000x reference)
import functools

import jax
import jax.numpy as jnp
from jax import lax
from jax.experimental import pallas as pl
from jax.experimental.pallas import tpu as pltpu
from jax.experimental.pallas import tpu_sc as plsc

N = 10000
E = 320000
D = 128
K = 4
L = 2

NC, NS = 2, 16
NW = NC * NS
EPW = E // NW
CH = 80
NCHUNK = EPW // CH
RPS = N // NS

_mesh = plsc.VectorSubcoreMesh(core_axis_name="c", subcore_axis_name="s")


@functools.partial(
    pl.kernel,
    out_type=jax.ShapeDtypeStruct((NC, N, D), jnp.float32),
    mesh=_mesh,
    scratch_types=[
        pltpu.VMEM((CH,), jnp.int32),
        pltpu.VMEM((CH,), jnp.int32),
        pltpu.VMEM((CH, D), jnp.float32),
        pltpu.VMEM((CH, D), jnp.float32),
        pltpu.VMEM_SHARED((N, D), jnp.float32),
        pltpu.SemaphoreType.DMA,
    ],
)
def _sc_aggregate(x_hbm, src_hbm, dst_hbm, ea_hbm, out_hbm,
                  src_v, dst_v, xg_v, ea_v, agg_sh, sem):
    cid = lax.axis_index("c")
    sid = lax.axis_index("s")
    wid = sid * NC + cid

    def _zrow(i, carry):
        for j in range(D // 16):
            xg_v[i, pl.ds(j * 16, 16)] = jnp.zeros((16,), jnp.float32)
        return carry
    lax.fori_loop(0, CH, _zrow, 0)
    r0 = sid * RPS
    nfull = RPS // CH
    for z in range(nfull):
        pltpu.sync_copy(xg_v, agg_sh.at[pl.ds(r0 + z * CH, CH)])
    rem = RPS - nfull * CH
    pltpu.sync_copy(xg_v.at[pl.ds(0, rem)],
                    agg_sh.at[pl.ds(r0 + nfull * CH, rem)])
    plsc.subcore_barrier()

    def _chunk(j, carry):
        base = wid * EPW + j * CH
        pltpu.sync_copy(src_hbm.at[pl.ds(base, CH)], src_v)
        pltpu.sync_copy(dst_hbm.at[pl.ds(base, CH)], dst_v)
        pltpu.async_copy(x_hbm.at[src_v], xg_v, sem).wait()
        pltpu.sync_copy(ea_hbm.at[pl.ds(base, CH)], ea_v)

        def _row(i, c2):
            for jj in range(D // 16):
                sl = pl.ds(jj * 16, 16)
                xg_v[i, sl] = jnp.maximum(xg_v[i, sl] + ea_v[i, sl], 0.0)
            return c2
        lax.fori_loop(0, CH, _row, 0)
        pltpu.sync_copy(xg_v, agg_sh.at[dst_v], add=True)
        return carry
    lax.fori_loop(0, NCHUNK, _chunk, 0)
    plsc.subcore_barrier()

    for z in range(nfull):
        rr = r0 + z * CH
        pltpu.sync_copy(agg_sh.at[pl.ds(rr, CH)],
                        out_hbm.at[cid, pl.ds(rr, CH)])
    rr = r0 + nfull * CH
    pltpu.sync_copy(agg_sh.at[pl.ds(rr, rem)],
                    out_hbm.at[cid, pl.ds(rr, rem)])


def _bn(h, g, b):
    m = jnp.mean(h, axis=0, keepdims=True)
    v = jnp.mean((h - m) * (h - m), axis=0, keepdims=True)
    return (h - m) * lax.rsqrt(v + 1e-5) * g + b


def _mask_body(enc_ref, w1_ref, b1_ref, g1_ref, bb1_ref,
               w2_ref, b2_ref, g2_ref, bb2_ref, o_ref):
    h = jnp.dot(enc_ref[...], w1_ref[...],
                preferred_element_type=jnp.float32) + b1_ref[...]
    h = jnp.maximum(_bn(h, g1_ref[...], bb1_ref[...]), 0.0)
    h = jnp.dot(h, w2_ref[...],
                preferred_element_type=jnp.float32) + b2_ref[...]
    h = jnp.maximum(_bn(h, g2_ref[...], bb2_ref[...]), 0.0)
    h = h - jnp.max(h, axis=1, keepdims=True)
    eh = jnp.exp(h)
    o_ref[...] = eh / jnp.sum(eh, axis=1, keepdims=True)


_mask_call = pl.pallas_call(
    _mask_body,
    out_shape=jax.ShapeDtypeStruct((N, K), jnp.float32),
)


def _dense_body(x_ref, a0_ref, a1_ref, xin_ref, m_ref, eps_ref,
                w1_ref, b1_ref, w2_ref, b2_ref, g_ref, bb_ref, o_ref,
                *, last):
    x = x_ref[...]
    hc = x * eps_ref[...] + (a0_ref[...] + a1_ref[...])
    h = jnp.maximum(
        jnp.dot(hc, w1_ref[...], preferred_element_type=jnp.float32)
        + b1_ref[...], 0.0)
    h = jnp.dot(h, w2_ref[...],
                preferred_element_type=jnp.float32) + b2_ref[...]
    xp = m_ref[...] * h + x
    xn = _bn(xp, g_ref[...], bb_ref[...])
    if last:
        xn = xin_ref[...] + jnp.maximum(xn, 0.0)
    o_ref[...] = xn


_dense_call = {
    last: pl.pallas_call(
        functools.partial(_dense_body, last=last),
        out_shape=jax.ShapeDtypeStruct((N, D), jnp.float32),
    )
    for last in (False, True)
}


def kernel(x, edge_index, edge_attr, encoding, cur_layer,
           me_W1, me_b1, me_bn1_g, me_bn1_b, me_W2, me_b2, me_bn2_g,
           me_bn2_b, gine_W1, gine_b1, gine_W2, gine_b2, gine_eps,
           bn_g, bn_b):
    src = edge_index[0]
    dst = edge_index[1]
    masks = _mask_call(encoding,
                       me_W1, me_b1.reshape(1, -1),
                       me_bn1_g.reshape(1, -1), me_bn1_b.reshape(1, -1),
                       me_W2, me_b2.reshape(1, -1),
                       me_bn2_g.reshape(1, -1), me_bn2_b.reshape(1, -1))
    for l in range(L):
        x_in = x
        for c in range(K):
            agg = _sc_aggregate(x, src, dst, edge_attr)
            epsv = jnp.full((1, D), 1.0 + gine_eps[l, c], jnp.float32)
            x = _dense_call[c == K - 1](
                x, agg[0], agg[1], x_in, masks[:, c:c + 1], epsv,
                gine_W1[l, c], gine_b1[l, c].reshape(1, -1),
                gine_W2[l, c], gine_b2[l, c].reshape(1, -1),
                bn_g[l, c].reshape(1, -1), bn_b[l, c].reshape(1, -1))
    return x


# speedup vs baseline: 2.8933x; 2.8933x over previous
import functools

import jax
import jax.numpy as jnp
from jax import lax
from jax.experimental import pallas as pl
from jax.experimental.pallas import tpu as pltpu
from jax.experimental.pallas import tpu_sc as plsc

N = 10000
E = 320000
D = 128
K = 4
L = 2

NC, NS = 2, 16
NW = NC * NS
EPW = E // NW
CH = 80
NCHUNK = EPW // CH
NRCH = N // CH
RCPS = -(-NRCH // NS)

_mesh = plsc.VectorSubcoreMesh(core_axis_name="c", subcore_axis_name="s",
                               num_cores=NC, num_subcores=NS)


@functools.partial(
    pl.kernel,
    out_type=jax.ShapeDtypeStruct((NC, N, D), jnp.float32),
    mesh=_mesh,
    scratch_types=[
        pltpu.VMEM((CH,), jnp.int32),
        pltpu.VMEM((CH,), jnp.int32),
        pltpu.VMEM((CH, D), jnp.float32),
        pltpu.VMEM((CH, D), jnp.float32),
        pltpu.VMEM_SHARED((N, D), jnp.float32),
        pltpu.SemaphoreType.DMA,
    ],
)
def _sc_aggregate(x_hbm, src_hbm, dst_hbm, ea_hbm, out_hbm,
                  src_v, dst_v, xg_v, ea_v, agg_sh, sem):
    cid = lax.axis_index("c")
    sid = lax.axis_index("s")
    wid = sid * NC + cid

    def _zrow(i, carry):
        for j in range(D // 16):
            xg_v[i, pl.ds(j * 16, 16)] = jnp.zeros((16,), jnp.float32)
        return carry
    lax.fori_loop(0, CH, _zrow, 0)
    for z in range(RCPS):
        rc = z * NS + sid

        @pl.when(rc < NRCH)
        def _():
            pltpu.sync_copy(xg_v, agg_sh.at[pl.ds(rc * CH, CH)])
    plsc.subcore_barrier()

    def _chunk(j, carry):
        base = wid * EPW + j * CH
        pltpu.sync_copy(src_hbm.at[pl.ds(base, CH)], src_v)
        pltpu.sync_copy(dst_hbm.at[pl.ds(base, CH)], dst_v)
        pltpu.async_copy(x_hbm.at[src_v], xg_v, sem).wait()
        pltpu.sync_copy(ea_hbm.at[pl.ds(base, CH)], ea_v)

        def _row(i, c2):
            for jj in range(D // 16):
                sl = pl.ds(jj * 16, 16)
                xg_v[i, sl] = jnp.maximum(xg_v[i, sl] + ea_v[i, sl], 0.0)
            return c2
        lax.fori_loop(0, CH, _row, 0)
        pltpu.sync_copy(xg_v, agg_sh.at[dst_v], add=True)
        return carry
    lax.fori_loop(0, NCHUNK, _chunk, 0)
    plsc.subcore_barrier()

    for z in range(RCPS):
        rc = z * NS + sid

        @pl.when(rc < NRCH)
        def _():
            pltpu.sync_copy(agg_sh.at[pl.ds(rc * CH, CH)],
                            out_hbm.at[cid, pl.ds(rc * CH, CH)])


def _bn(h, g, b):
    m = jnp.mean(h, axis=0, keepdims=True)
    v = jnp.mean((h - m) * (h - m), axis=0, keepdims=True)
    return (h - m) * lax.rsqrt(v + 1e-5) * g + b


def _mask_body(enc_ref, w1_ref, b1_ref, g1_ref, bb1_ref,
               w2_ref, b2_ref, g2_ref, bb2_ref, o_ref):
    h = jnp.dot(enc_ref[...], w1_ref[...],
                preferred_element_type=jnp.float32) + b1_ref[...]
    h = jnp.maximum(_bn(h, g1_ref[...], bb1_ref[...]), 0.0)
    h = jnp.dot(h, w2_ref[...],
                preferred_element_type=jnp.float32) + b2_ref[...]
    h = jnp.maximum(_bn(h, g2_ref[...], bb2_ref[...]), 0.0)
    h = h - jnp.max(h, axis=1, keepdims=True)
    eh = jnp.exp(h)
    o_ref[...] = eh / jnp.sum(eh, axis=1, keepdims=True)


_mask_call = pl.pallas_call(
    _mask_body,
    out_shape=jax.ShapeDtypeStruct((N, K), jnp.float32),
)


def _dense_body(x_ref, a0_ref, a1_ref, xin_ref, m_ref, eps_ref,
                w1_ref, b1_ref, w2_ref, b2_ref, g_ref, bb_ref, o_ref,
                *, last):
    x = x_ref[...]
    hc = x * eps_ref[...] + (a0_ref[...] + a1_ref[...])
    h = jnp.maximum(
        jnp.dot(hc, w1_ref[...], preferred_element_type=jnp.float32)
        + b1_ref[...], 0.0)
    h = jnp.dot(h, w2_ref[...],
                preferred_element_type=jnp.float32) + b2_ref[...]
    xp = m_ref[...] * h + x
    xn = _bn(xp, g_ref[...], bb_ref[...])
    if last:
        xn = xin_ref[...] + jnp.maximum(xn, 0.0)
    o_ref[...] = xn


_dense_call = {
    last: pl.pallas_call(
        functools.partial(_dense_body, last=last),
        out_shape=jax.ShapeDtypeStruct((N, D), jnp.float32),
    )
    for last in (False, True)
}


def kernel(x, edge_index, edge_attr, encoding, cur_layer,
           me_W1, me_b1, me_bn1_g, me_bn1_b, me_W2, me_b2, me_bn2_g,
           me_bn2_b, gine_W1, gine_b1, gine_W2, gine_b2, gine_eps,
           bn_g, bn_b):
    src = edge_index[0]
    dst = edge_index[1]
    masks = _mask_call(encoding,
                       me_W1, me_b1.reshape(1, -1),
                       me_bn1_g.reshape(1, -1), me_bn1_b.reshape(1, -1),
                       me_W2, me_b2.reshape(1, -1),
                       me_bn2_g.reshape(1, -1), me_bn2_b.reshape(1, -1))
    for l in range(L):
        x_in = x
        for c in range(K):
            agg = _sc_aggregate(x, src, dst, edge_attr)
            epsv = jnp.full((1, D), 1.0 + gine_eps[l, c], jnp.float32)
            x = _dense_call[c == K - 1](
                x, agg[0], agg[1], x_in, masks[:, c:c + 1], epsv,
                gine_W1[l, c], gine_b1[l, c].reshape(1, -1),
                gine_W2[l, c], gine_b2[l, c].reshape(1, -1),
                bn_g[l, c].reshape(1, -1), bn_b[l, c].reshape(1, -1))
    return x


# speedup vs baseline: 5.3435x; 1.8469x over previous
import functools

import jax
import jax.numpy as jnp
from jax import lax
from jax.experimental import pallas as pl
from jax.experimental.pallas import tpu as pltpu
from jax.experimental.pallas import tpu_sc as plsc

N = 10000
E = 320000
D = 128
K = 4
L = 2

NC, NS = 2, 16
NW = NC * NS
EPW = E // NW
CH = 80
NCHUNK = EPW // CH
NRCH = N // CH
RCPS = -(-NRCH // NS)

_mesh = plsc.VectorSubcoreMesh(core_axis_name="c", subcore_axis_name="s",
                               num_cores=NC, num_subcores=NS)


@functools.partial(
    pl.kernel,
    out_type=jax.ShapeDtypeStruct((NC, N, D), jnp.float32),
    mesh=_mesh,
    scratch_types=[
        [pltpu.VMEM((CH,), jnp.int32) for _ in range(2)],
        [pltpu.VMEM((CH,), jnp.int32) for _ in range(2)],
        [pltpu.VMEM((CH, D), jnp.float32) for _ in range(2)],
        [pltpu.VMEM((CH, D), jnp.float32) for _ in range(2)],
        pltpu.VMEM_SHARED((N, D), jnp.float32),
        [pltpu.SemaphoreType.DMA for _ in range(2)],
        [pltpu.SemaphoreType.DMA for _ in range(2)],
    ],
)
def _sc_aggregate(x_hbm, src_hbm, dst_hbm, ea_hbm, out_hbm,
                  src_v, dst_v, xg_v, ea_v, agg_sh, gsem, esem):
    cid = lax.axis_index("c")
    sid = lax.axis_index("s")
    wid = sid * NC + cid

    def _zrow(i, carry):
        for j in range(D // 16):
            xg_v[0][i, pl.ds(j * 16, 16)] = jnp.zeros((16,), jnp.float32)
        return carry
    lax.fori_loop(0, CH, _zrow, 0)
    for z in range(RCPS):
        rc = z * NS + sid

        @pl.when(rc < NRCH)
        def _():
            pltpu.sync_copy(xg_v[0], agg_sh.at[pl.ds(rc * CH, CH)])
    plsc.subcore_barrier()

    def _prefetch(j, b):
        base = wid * EPW + j * CH
        pltpu.sync_copy(src_hbm.at[pl.ds(base, CH)], src_v[b])
        pltpu.sync_copy(dst_hbm.at[pl.ds(base, CH)], dst_v[b])
        pltpu.async_copy(x_hbm.at[src_v[b]], xg_v[b], gsem[b])
        pltpu.async_copy(ea_hbm.at[pl.ds(base, CH)], ea_v[b], esem[b])

    def _process(b):
        pltpu.make_async_copy(x_hbm.at[src_v[b]], xg_v[b], gsem[b]).wait()
        pltpu.make_async_copy(ea_hbm.at[pl.ds(0, CH)], ea_v[b],
                              esem[b]).wait()

        def _row(i, c2):
            for jj in range(D // 16):
                sl = pl.ds(jj * 16, 16)
                xg_v[b][i, sl] = jnp.maximum(
                    xg_v[b][i, sl] + ea_v[b][i, sl], 0.0)
            return c2
        lax.fori_loop(0, CH, _row, 0)
        pltpu.sync_copy(xg_v[b], agg_sh.at[dst_v[b]], add=True)

    _prefetch(0, 0)
    npair = NCHUNK // 2 + 1

    def _pair(t, carry):
        @pl.when(t < npair - 1)
        def _():
            _prefetch(2 * t + 1, 1)
        _process(0)

        @pl.when(t < npair - 1)
        def _():
            _prefetch(2 * t + 2, 0)
            _process(1)
        return carry
    lax.fori_loop(0, npair, _pair, 0)
    plsc.subcore_barrier()

    for z in range(RCPS):
        rc = z * NS + sid

        @pl.when(rc < NRCH)
        def _():
            pltpu.sync_copy(agg_sh.at[pl.ds(rc * CH, CH)],
                            out_hbm.at[cid, pl.ds(rc * CH, CH)])


def _bn(h, g, b):
    m = jnp.mean(h, axis=0, keepdims=True)
    v = jnp.mean((h - m) * (h - m), axis=0, keepdims=True)
    return (h - m) * lax.rsqrt(v + 1e-5) * g + b


def _mask_body(enc_ref, w1_ref, b1_ref, g1_ref, bb1_ref,
               w2_ref, b2_ref, g2_ref, bb2_ref, o_ref):
    h = jnp.dot(enc_ref[...], w1_ref[...],
                preferred_element_type=jnp.float32) + b1_ref[...]
    h = jnp.maximum(_bn(h, g1_ref[...], bb1_ref[...]), 0.0)
    h = jnp.dot(h, w2_ref[...],
                preferred_element_type=jnp.float32) + b2_ref[...]
    h = jnp.maximum(_bn(h, g2_ref[...], bb2_ref[...]), 0.0)
    h = h - jnp.max(h, axis=1, keepdims=True)
    eh = jnp.exp(h)
    o_ref[...] = eh / jnp.sum(eh, axis=1, keepdims=True)


_mask_call = pl.pallas_call(
    _mask_body,
    out_shape=jax.ShapeDtypeStruct((N, K), jnp.float32),
)


def _dense_body(x_ref, a0_ref, a1_ref, xin_ref, m_ref, eps_ref,
                w1_ref, b1_ref, w2_ref, b2_ref, g_ref, bb_ref, o_ref,
                *, last):
    x = x_ref[...]
    hc = x * eps_ref[...] + (a0_ref[...] + a1_ref[...])
    h = jnp.maximum(
        jnp.dot(hc, w1_ref[...], preferred_element_type=jnp.float32)
        + b1_ref[...], 0.0)
    h = jnp.dot(h, w2_ref[...],
                preferred_element_type=jnp.float32) + b2_ref[...]
    xp = m_ref[...] * h + x
    xn = _bn(xp, g_ref[...], bb_ref[...])
    if last:
        xn = xin_ref[...] + jnp.maximum(xn, 0.0)
    o_ref[...] = xn


_dense_call = {
    last: pl.pallas_call(
        functools.partial(_dense_body, last=last),
        out_shape=jax.ShapeDtypeStruct((N, D), jnp.float32),
    )
    for last in (False, True)
}


def kernel(x, edge_index, edge_attr, encoding, cur_layer,
           me_W1, me_b1, me_bn1_g, me_bn1_b, me_W2, me_b2, me_bn2_g,
           me_bn2_b, gine_W1, gine_b1, gine_W2, gine_b2, gine_eps,
           bn_g, bn_b):
    src = edge_index[0]
    dst = edge_index[1]
    masks = _mask_call(encoding,
                       me_W1, me_b1.reshape(1, -1),
                       me_bn1_g.reshape(1, -1), me_bn1_b.reshape(1, -1),
                       me_W2, me_b2.reshape(1, -1),
                       me_bn2_g.reshape(1, -1), me_bn2_b.reshape(1, -1))
    for l in range(L):
        x_in = x
        for c in range(K):
            agg = _sc_aggregate(x, src, dst, edge_attr)
            epsv = jnp.full((1, D), 1.0 + gine_eps[l, c], jnp.float32)
            x = _dense_call[c == K - 1](
                x, agg[0], agg[1], x_in, masks[:, c:c + 1], epsv,
                gine_W1[l, c], gine_b1[l, c].reshape(1, -1),
                gine_W2[l, c], gine_b2[l, c].reshape(1, -1),
                bn_g[l, c].reshape(1, -1), bn_b[l, c].reshape(1, -1))
    return x


# speedup vs baseline: 5.5579x; 1.0401x over previous
import functools

import jax
import jax.numpy as jnp
from jax import lax
from jax.experimental import pallas as pl
from jax.experimental.pallas import tpu as pltpu
from jax.experimental.pallas import tpu_sc as plsc

N = 10000
E = 320000
D = 128
K = 4
L = 2

NC, NS = 2, 16
NW = NC * NS
EPW = E // NW
CH = 40
NCHUNK = EPW // CH
NRCH = N // CH
RCPS = -(-NRCH // NS)

_mesh = plsc.VectorSubcoreMesh(core_axis_name="c", subcore_axis_name="s",
                               num_cores=NC, num_subcores=NS)


@functools.partial(
    pl.kernel,
    out_type=jax.ShapeDtypeStruct((NC, N, D), jnp.float32),
    mesh=_mesh,
    scratch_types=[
        pltpu.VMEM((EPW,), jnp.int32),
        [pltpu.VMEM((CH,), jnp.int32) for _ in range(2)],
        [pltpu.VMEM((CH, D), jnp.float32) for _ in range(2)],
        [pltpu.VMEM((CH, D), jnp.float32) for _ in range(2)],
        pltpu.VMEM_SHARED((N, D), jnp.float32),
        [pltpu.SemaphoreType.DMA for _ in range(2)],
        [pltpu.SemaphoreType.DMA for _ in range(2)],
        [pltpu.SemaphoreType.DMA for _ in range(2)],
        [pltpu.SemaphoreType.DMA for _ in range(2)],
    ],
)
def _sc_aggregate(x_hbm, src_hbm, dst_hbm, ea_hbm, out_hbm,
                  src_v, dst_v, xg_v, ea_v, agg_sh, gsem, esem, ssem, dsem):
    cid = lax.axis_index("c")
    sid = lax.axis_index("s")
    wid = sid * NC + cid

    def _zrow(i, carry):
        for j in range(D // 16):
            xg_v[0][i, pl.ds(j * 16, 16)] = jnp.zeros((16,), jnp.float32)
        return carry
    lax.fori_loop(0, CH, _zrow, 0)
    for z in range(RCPS):
        rc = z * NS + sid

        @pl.when(rc < NRCH)
        def _():
            pltpu.sync_copy(xg_v[0], agg_sh.at[pl.ds(rc * CH, CH)])

    pltpu.sync_copy(src_hbm.at[pl.ds(wid * EPW, EPW)], src_v)

    def _fetch(j, b):
        base = wid * EPW + j * CH
        pltpu.async_copy(x_hbm.at[src_v.at[pl.ds(j * CH, CH)]],
                         xg_v[b], gsem[b])
        pltpu.async_copy(ea_hbm.at[pl.ds(base, CH)], ea_v[b], esem[b])
        pltpu.async_copy(dst_hbm.at[pl.ds(base, CH)], dst_v[b], dsem[b])

    _fetch(0, 0)
    plsc.subcore_barrier()

    def _wait_scatter(b):
        pltpu.make_async_copy(xg_v[b], agg_sh.at[dst_v[b]],
                              ssem[b]).wait()

    def _step(t, j, b, nb):
        pltpu.make_async_copy(x_hbm.at[src_v.at[pl.ds(0, CH)]], xg_v[b],
                              gsem[b]).wait()
        pltpu.make_async_copy(ea_hbm.at[pl.ds(0, CH)], ea_v[b],
                              esem[b]).wait()
        pltpu.make_async_copy(dst_hbm.at[pl.ds(0, CH)], dst_v[b],
                              dsem[b]).wait()

        @pl.when(t > 0)
        def _():
            _wait_scatter(nb)

        @pl.when(j + 1 < NCHUNK)
        def _():
            _fetch(j + 1, nb)

        def _row(i, c2):
            for jj in range(D // 16):
                sl = pl.ds(jj * 16, 16)
                xg_v[b][i, sl] = jnp.maximum(
                    xg_v[b][i, sl] + ea_v[b][i, sl], 0.0)
            return c2
        lax.fori_loop(0, CH, _row, 0)
        pltpu.async_copy(xg_v[b], agg_sh.at[dst_v[b]], ssem[b],
                         add=True)

    npair = NCHUNK // 2

    def _pair(t, carry):
        _step(t, 2 * t, 0, 1)
        _step(2 * t + 1, 2 * t + 1, 1, 0)
        return carry
    lax.fori_loop(0, npair, _pair, 0)
    _wait_scatter(1)
    plsc.subcore_barrier()

    for z in range(RCPS):
        rc = z * NS + sid

        @pl.when(rc < NRCH)
        def _():
            pltpu.sync_copy(agg_sh.at[pl.ds(rc * CH, CH)],
                            out_hbm.at[cid, pl.ds(rc * CH, CH)])


def _bn(h, g, b):
    m = jnp.mean(h, axis=0, keepdims=True)
    v = jnp.mean((h - m) * (h - m), axis=0, keepdims=True)
    return (h - m) * lax.rsqrt(v + 1e-5) * g + b


def _mask_body(enc_ref, w1_ref, b1_ref, g1_ref, bb1_ref,
               w2_ref, b2_ref, g2_ref, bb2_ref, o_ref):
    h = jnp.dot(enc_ref[...], w1_ref[...],
                preferred_element_type=jnp.float32) + b1_ref[...]
    h = jnp.maximum(_bn(h, g1_ref[...], bb1_ref[...]), 0.0)
    h = jnp.dot(h, w2_ref[...],
                preferred_element_type=jnp.float32) + b2_ref[...]
    h = jnp.maximum(_bn(h, g2_ref[...], bb2_ref[...]), 0.0)
    h = h - jnp.max(h, axis=1, keepdims=True)
    eh = jnp.exp(h)
    o_ref[...] = eh / jnp.sum(eh, axis=1, keepdims=True)


_mask_call = pl.pallas_call(
    _mask_body,
    out_shape=jax.ShapeDtypeStruct((N, K), jnp.float32),
)


def _dense_body(x_ref, a0_ref, a1_ref, xin_ref, m_ref, eps_ref,
                w1_ref, b1_ref, w2_ref, b2_ref, g_ref, bb_ref, o_ref,
                *, last):
    x = x_ref[...]
    hc = x * eps_ref[...] + (a0_ref[...] + a1_ref[...])
    h = jnp.maximum(
        jnp.dot(hc, w1_ref[...], preferred_element_type=jnp.float32)
        + b1_ref[...], 0.0)
    h = jnp.dot(h, w2_ref[...],
                preferred_element_type=jnp.float32) + b2_ref[...]
    xp = m_ref[...] * h + x
    xn = _bn(xp, g_ref[...], bb_ref[...])
    if last:
        xn = xin_ref[...] + jnp.maximum(xn, 0.0)
    o_ref[...] = xn


_dense_call = {
    last: pl.pallas_call(
        functools.partial(_dense_body, last=last),
        out_shape=jax.ShapeDtypeStruct((N, D), jnp.float32),
    )
    for last in (False, True)
}


def kernel(x, edge_index, edge_attr, encoding, cur_layer,
           me_W1, me_b1, me_bn1_g, me_bn1_b, me_W2, me_b2, me_bn2_g,
           me_bn2_b, gine_W1, gine_b1, gine_W2, gine_b2, gine_eps,
           bn_g, bn_b):
    src = edge_index[0]
    dst = edge_index[1]
    masks = _mask_call(encoding,
                       me_W1, me_b1.reshape(1, -1),
                       me_bn1_g.reshape(1, -1), me_bn1_b.reshape(1, -1),
                       me_W2, me_b2.reshape(1, -1),
                       me_bn2_g.reshape(1, -1), me_bn2_b.reshape(1, -1))
    for l in range(L):
        x_in = x
        for c in range(K):
            agg = _sc_aggregate(x, src, dst, edge_attr)
            epsv = jnp.full((1, D), 1.0 + gine_eps[l, c], jnp.float32)
            x = _dense_call[c == K - 1](
                x, agg[0], agg[1], x_in, masks[:, c:c + 1], epsv,
                gine_W1[l, c], gine_b1[l, c].reshape(1, -1),
                gine_W2[l, c], gine_b2[l, c].reshape(1, -1),
                bn_g[l, c].reshape(1, -1), bn_b[l, c].reshape(1, -1))
    return x


# speedup vs baseline: 7.8758x; 1.4170x over previous
import functools

import jax
import jax.numpy as jnp
from jax import lax
from jax.experimental import pallas as pl
from jax.experimental.pallas import tpu as pltpu
from jax.experimental.pallas import tpu_sc as plsc

N = 10000
E = 320000
D = 128
K = 4
L = 2

NC, NS = 2, 16
NW = NC * NS
EPW = E // NW
CH = 40
NCHUNK = EPW // CH
NRCH = N // CH
RCPS = -(-NRCH // NS)

_mesh = plsc.VectorSubcoreMesh(core_axis_name="c", subcore_axis_name="s",
                               num_cores=NC, num_subcores=NS)


@functools.partial(
    pl.kernel,
    out_type=jax.ShapeDtypeStruct((NC, N, D), jnp.float32),
    mesh=_mesh,
    scratch_types=[
        pltpu.VMEM((EPW,), jnp.int32),
        [pltpu.VMEM((CH,), jnp.int32) for _ in range(3)],
        [pltpu.VMEM((CH, D), jnp.float32) for _ in range(3)],
        [pltpu.VMEM((CH, D), jnp.float32) for _ in range(3)],
        pltpu.VMEM_SHARED((N, D), jnp.float32),
        [pltpu.SemaphoreType.DMA for _ in range(3)],
        [pltpu.SemaphoreType.DMA for _ in range(3)],
        [pltpu.SemaphoreType.DMA for _ in range(3)],
        [pltpu.SemaphoreType.DMA for _ in range(3)],
    ],
)
def _sc_aggregate(x_hbm, src_hbm, dst_hbm, ea_hbm, out_hbm,
                  src_v, dst_v, xg_v, ea_v, agg_sh, gsem, esem, ssem, dsem):
    cid = lax.axis_index("c")
    sid = lax.axis_index("s")
    wid = sid * NC + cid

    def _zrow(i, carry):
        for j in range(D // 16):
            xg_v[0][i, pl.ds(j * 16, 16)] = jnp.zeros((16,), jnp.float32)
        return carry
    lax.fori_loop(0, CH, _zrow, 0)
    for z in range(RCPS):
        rc = z * NS + sid

        @pl.when(rc < NRCH)
        def _():
            pltpu.sync_copy(xg_v[0], agg_sh.at[pl.ds(rc * CH, CH)])

    pltpu.sync_copy(src_hbm.at[pl.ds(wid * EPW, EPW)], src_v)

    def _fetch(j, b):
        base = wid * EPW + j * CH
        pltpu.async_copy(dst_hbm.at[pl.ds(base, CH)], dst_v[b], dsem[b])
        pltpu.async_copy(x_hbm.at[src_v.at[pl.ds(j * CH, CH)]],
                         xg_v[b], gsem[b])
        pltpu.async_copy(ea_hbm.at[pl.ds(base, CH)], ea_v[b], esem[b])

    _fetch(0, 0)
    _fetch(1, 1)
    plsc.subcore_barrier()

    def _wait_scatter(b):
        pltpu.make_async_copy(xg_v[b], agg_sh.at[dst_v[b]],
                              ssem[b]).wait()

    def _step(j, b, b2):
        pltpu.make_async_copy(x_hbm.at[src_v.at[pl.ds(0, CH)]], xg_v[b],
                              gsem[b]).wait()
        pltpu.make_async_copy(ea_hbm.at[pl.ds(0, CH)], ea_v[b],
                              esem[b]).wait()

        @pl.when(j >= 1)
        def _():
            _wait_scatter(b2)

        @pl.when(j + 2 < NCHUNK)
        def _():
            _fetch(j + 2, b2)

        pltpu.make_async_copy(dst_hbm.at[pl.ds(0, CH)], dst_v[b],
                              dsem[b]).wait()

        def _row(i, c2):
            for jj in range(D // 16):
                sl = pl.ds(jj * 16, 16)
                xg_v[b][i, sl] = jnp.maximum(
                    xg_v[b][i, sl] + ea_v[b][i, sl], 0.0)
            return c2
        lax.fori_loop(0, CH, _row, 0)
        pltpu.async_copy(xg_v[b], agg_sh.at[dst_v[b]], ssem[b],
                         add=True)

    ntri = (NCHUNK - 1) // 3

    def _tri(t, carry):
        _step(3 * t, 0, 2)
        _step(3 * t + 1, 1, 0)
        _step(3 * t + 2, 2, 1)
        return carry
    lax.fori_loop(0, ntri, _tri, 0)
    _step(NCHUNK - 1, 0, 2)
    _wait_scatter(0)
    plsc.subcore_barrier()

    for z in range(RCPS):
        rc = z * NS + sid

        @pl.when(rc < NRCH)
        def _():
            pltpu.sync_copy(agg_sh.at[pl.ds(rc * CH, CH)],
                            out_hbm.at[cid, pl.ds(rc * CH, CH)])


def _bn(h, g, b):
    m = jnp.mean(h, axis=0, keepdims=True)
    v = jnp.mean((h - m) * (h - m), axis=0, keepdims=True)
    return (h - m) * lax.rsqrt(v + 1e-5) * g + b


def _mask_body(enc_ref, w1_ref, b1_ref, g1_ref, bb1_ref,
               w2_ref, b2_ref, g2_ref, bb2_ref, o_ref):
    h = jnp.dot(enc_ref[...], w1_ref[...],
                preferred_element_type=jnp.float32) + b1_ref[...]
    h = jnp.maximum(_bn(h, g1_ref[...], bb1_ref[...]), 0.0)
    h = jnp.dot(h, w2_ref[...],
                preferred_element_type=jnp.float32) + b2_ref[...]
    h = jnp.maximum(_bn(h, g2_ref[...], bb2_ref[...]), 0.0)
    h = h - jnp.max(h, axis=1, keepdims=True)
    eh = jnp.exp(h)
    o_ref[...] = eh / jnp.sum(eh, axis=1, keepdims=True)


_mask_call = pl.pallas_call(
    _mask_body,
    out_shape=jax.ShapeDtypeStruct((N, K), jnp.float32),
)


def _dense_body(x_ref, a0_ref, a1_ref, xin_ref, m_ref, eps_ref,
                w1_ref, b1_ref, w2_ref, b2_ref, g_ref, bb_ref, o_ref,
                *, last):
    x = x_ref[...]
    hc = x * eps_ref[...] + (a0_ref[...] + a1_ref[...])
    h = jnp.maximum(
        jnp.dot(hc, w1_ref[...], preferred_element_type=jnp.float32)
        + b1_ref[...], 0.0)
    h = jnp.dot(h, w2_ref[...],
                preferred_element_type=jnp.float32) + b2_ref[...]
    xp = m_ref[...] * h + x
    xn = _bn(xp, g_ref[...], bb_ref[...])
    if last:
        xn = xin_ref[...] + jnp.maximum(xn, 0.0)
    o_ref[...] = xn


_dense_call = {
    last: pl.pallas_call(
        functools.partial(_dense_body, last=last),
        out_shape=jax.ShapeDtypeStruct((N, D), jnp.float32),
    )
    for last in (False, True)
}


def kernel(x, edge_index, edge_attr, encoding, cur_layer,
           me_W1, me_b1, me_bn1_g, me_bn1_b, me_W2, me_b2, me_bn2_g,
           me_bn2_b, gine_W1, gine_b1, gine_W2, gine_b2, gine_eps,
           bn_g, bn_b):
    src = edge_index[0]
    dst = edge_index[1]
    masks = _mask_call(encoding,
                       me_W1, me_b1.reshape(1, -1),
                       me_bn1_g.reshape(1, -1), me_bn1_b.reshape(1, -1),
                       me_W2, me_b2.reshape(1, -1),
                       me_bn2_g.reshape(1, -1), me_bn2_b.reshape(1, -1))
    for l in range(L):
        x_in = x
        for c in range(K):
            agg = _sc_aggregate(x, src, dst, edge_attr)
            epsv = jnp.full((1, D), 1.0 + gine_eps[l, c], jnp.float32)
            x = _dense_call[c == K - 1](
                x, agg[0], agg[1], x_in, masks[:, c:c + 1], epsv,
                gine_W1[l, c], gine_b1[l, c].reshape(1, -1),
                gine_W2[l, c], gine_b2[l, c].reshape(1, -1),
                bn_g[l, c].reshape(1, -1), bn_b[l, c].reshape(1, -1))
    return x
